# baseline (speedup 1.0000x reference)
import jax
import jax.numpy as jnp
from jax import lax
from jax.experimental import pallas as pl
from jax.experimental.pallas import tpu as pltpu
from jax.experimental.pallas import tpu_sc as plsc

NUM_EMB = 1000
EMB_DIM = 128
BATCH = 16384
LANES = 16
EPS = 1e-6
MARGIN = 1.0

_info = plsc.get_sparse_core_info()
_NC, _NS = _info.num_cores, _info.num_subcores
NW = _NC * _NS
B_PER_W = BATCH // NW
N_VECS = B_PER_W // LANES
N_STREAMS = B_PER_W // 128


V_PAD = 1024
N_CT = V_PAD // EMB_DIM


def _tc_gram_body(tf_ref, d2_ref):
    tf = tf_ref[...]
    n = jnp.sum(tf * tf, axis=1, keepdims=True)
    s = jnp.sum(tf, axis=1, keepdims=True)
    m = n + (2.0 * EPS) * s + EMB_DIM * EPS * EPS
    tfp = jnp.pad(tf, ((0, V_PAD - NUM_EMB), (0, 0)))
    g = lax.dot_general(tf, tfp, (((1,), (1,)), ((), ())),
                        preferred_element_type=jnp.float32)
    tt = jnp.transpose(tfp)
    w = (jnp.sum(tt * tt, axis=0, keepdims=True)
         - (2.0 * EPS) * jnp.sum(tt, axis=0, keepdims=True))
    for t in range(N_CT):
        lo, hi = t * EMB_DIM, (t + 1) * EMB_DIM
        d2_ref[t] = m + w[:, lo:hi] - 2.0 * g[:, lo:hi]


_tc_gram = pl.pallas_call(
    _tc_gram_body,
    out_shape=jax.ShapeDtypeStruct((N_CT, NUM_EMB, EMB_DIM), jnp.float32),
    in_specs=[pl.BlockSpec(memory_space=pltpu.VMEM)],
    out_specs=pl.BlockSpec(memory_space=pltpu.VMEM),
)


def _rsqrt_newton(x):
    i = lax.bitcast_convert_type(x, jnp.int32)
    r = lax.bitcast_convert_type(jnp.int32(0x5F3759DF) - (i >> 1), jnp.float32)
    hx = 0.5 * x
    for _ in range(3):
        r = r * (1.5 - hx * r * r)
    return r


def _sc_body(d2_hbm, a_hbm, p_hbm, n_hbm, out_hbm,
             idx_a, idx_p, idx_n, fa_ap, fa_an, g_ap, g_an, accv, sem):
    wid = lax.axis_index("s") * _NC + lax.axis_index("c")
    base = wid * B_PER_W

    ca = pltpu.async_copy(a_hbm.at[pl.ds(base, B_PER_W)], idx_a, sem)
    cp = pltpu.async_copy(p_hbm.at[pl.ds(base, B_PER_W)], idx_p, sem)
    cn = pltpu.async_copy(n_hbm.at[pl.ds(base, B_PER_W)], idx_n, sem)
    ca.wait()
    cp.wait()
    cn.wait()

    @plsc.parallel_loop(0, N_VECS, step=1, unroll=2)
    def fa_body(t):
        sl = pl.ds(t * LANES, LANES)
        abase = idx_a[sl] << 7
        p = idx_p[sl]
        fa_ap[sl] = (p >> 7) * (NUM_EMB * EMB_DIM) + abase + (p & 127)
        n = idx_n[sl]
        fa_an[sl] = (n >> 7) * (NUM_EMB * EMB_DIM) + abase + (n & 127)

    descs = []
    for t in range(N_STREAMS):
        sl = pl.ds(t * 128, 128)
        descs.append(pltpu.async_copy(d2_hbm.at[fa_ap.at[sl]], g_ap.at[sl], sem))
        descs.append(pltpu.async_copy(d2_hbm.at[fa_an.at[sl]], g_an.at[sl], sem))
    for d in descs:
        d.wait()

    def loss_body(t, acc):
        sl = pl.ds(t * LANES, LANES)
        x_ap = jnp.maximum(g_ap[sl], 1e-12)
        x_an = jnp.maximum(g_an[sl], 1e-12)
        d_ap = x_ap * _rsqrt_newton(x_ap)
        d_an = x_an * _rsqrt_newton(x_an)
        return acc + jnp.maximum(d_ap - d_an + MARGIN, 0.0)

    acc = lax.fori_loop(0, N_VECS, loss_body, jnp.zeros((LANES,), jnp.float32),
                        unroll=4)
    accv[...] = acc
    pltpu.sync_copy(accv, out_hbm.at[pl.ds(wid * LANES, LANES)])


_sc_pair_loss = pl.kernel(
    _sc_body,
    mesh=plsc.VectorSubcoreMesh(core_axis_name="c", subcore_axis_name="s"),
    compiler_params=pltpu.CompilerParams(use_tc_tiling_on_sc=False),
    out_type=jax.ShapeDtypeStruct((NW * LANES,), jnp.float32),
    scratch_types=[
        pltpu.VMEM((B_PER_W,), jnp.int32),
        pltpu.VMEM((B_PER_W,), jnp.int32),
        pltpu.VMEM((B_PER_W,), jnp.int32),
        pltpu.VMEM((B_PER_W,), jnp.int32),
        pltpu.VMEM((B_PER_W,), jnp.int32),
        pltpu.VMEM((B_PER_W,), jnp.float32),
        pltpu.VMEM((B_PER_W,), jnp.float32),
        pltpu.VMEM((LANES,), jnp.float32),
        pltpu.SemaphoreType.DMA,
    ],
)


def kernel(anchor, positive, negative, table):
    d2 = _tc_gram(table)
    d2_flat = d2.reshape(-1)
    parts = _sc_pair_loss(
        d2_flat, anchor.astype(jnp.int32), positive.astype(jnp.int32),
        negative.astype(jnp.int32))
    return jnp.sum(parts) / BATCH


# speedup vs baseline: 1.0052x; 1.0052x over previous
import jax
import jax.numpy as jnp
from jax import lax
from jax.experimental import pallas as pl
from jax.experimental.pallas import tpu as pltpu
from jax.experimental.pallas import tpu_sc as plsc

NUM_EMB = 1000
EMB_DIM = 128
BATCH = 16384
LANES = 16
EPS = 1e-6
MARGIN = 1.0

_info = plsc.get_sparse_core_info()
_NC, _NS = _info.num_cores, _info.num_subcores
NW = _NC * _NS
B_PER_W = BATCH // NW
N_VECS = B_PER_W // LANES
N_STREAMS = B_PER_W // 128


V_PAD = 1024
N_CT = V_PAD // EMB_DIM


def _tc_gram_body(tf_ref, d2_ref):
    tf = tf_ref[...]
    n = jnp.sum(tf * tf, axis=1, keepdims=True)
    s = jnp.sum(tf, axis=1, keepdims=True)
    m = n + (2.0 * EPS) * s + EMB_DIM * EPS * EPS
    tfp = jnp.pad(tf, ((0, V_PAD - NUM_EMB), (0, 0)))
    g = lax.dot_general(tf, tfp, (((1,), (1,)), ((), ())),
                        preferred_element_type=jnp.float32)
    tt = jnp.transpose(tfp)
    w = (jnp.sum(tt * tt, axis=0, keepdims=True)
         - (2.0 * EPS) * jnp.sum(tt, axis=0, keepdims=True))
    for t in range(N_CT):
        lo, hi = t * EMB_DIM, (t + 1) * EMB_DIM
        d2_ref[t] = m + w[:, lo:hi] - 2.0 * g[:, lo:hi]


_tc_gram = pl.pallas_call(
    _tc_gram_body,
    out_shape=jax.ShapeDtypeStruct((N_CT, NUM_EMB, EMB_DIM), jnp.float32),
    in_specs=[pl.BlockSpec(memory_space=pltpu.VMEM)],
    out_specs=pl.BlockSpec(memory_space=pltpu.VMEM),
)


def _rsqrt_newton(x):
    i = lax.bitcast_convert_type(x, jnp.int32)
    r = lax.bitcast_convert_type(jnp.int32(0x5F3759DF) - (i >> 1), jnp.float32)
    hx = 0.5 * x
    for _ in range(3):
        r = r * (1.5 - hx * r * r)
    return r


def _sc_body(d2_hbm, a_hbm, p_hbm, n_hbm, out_hbm,
             idx_a, idx_p, idx_n, fa_ap, fa_an, g_ap, g_an, accv, sem):
    wid = lax.axis_index("s") * _NC + lax.axis_index("c")
    base = wid * B_PER_W

    ca = pltpu.async_copy(a_hbm.at[pl.ds(base, B_PER_W)], idx_a, sem)
    cp = pltpu.async_copy(p_hbm.at[pl.ds(base, B_PER_W)], idx_p, sem)
    cn = pltpu.async_copy(n_hbm.at[pl.ds(base, B_PER_W)], idx_n, sem)
    ca.wait()
    cp.wait()
    cn.wait()

    @plsc.parallel_loop(0, N_VECS, step=1, unroll=2)
    def fa_body(t):
        sl = pl.ds(t * LANES, LANES)
        abase = idx_a[sl] << 7
        p = idx_p[sl]
        fa_ap[sl] = (p >> 7) * (NUM_EMB * EMB_DIM) + abase + (p & 127)
        n = idx_n[sl]
        fa_an[sl] = (n >> 7) * (NUM_EMB * EMB_DIM) + abase + (n & 127)

    descs = []
    for t in range(N_STREAMS):
        sl = pl.ds(t * 128, 128)
        descs.append(pltpu.async_copy(d2_hbm.at[fa_ap.at[sl]], g_ap.at[sl], sem))
        descs.append(pltpu.async_copy(d2_hbm.at[fa_an.at[sl]], g_an.at[sl], sem))
    for d in descs:
        d.wait()

    def loss_body(t, acc):
        sl = pl.ds(t * LANES, LANES)
        x_ap = jnp.maximum(g_ap[sl], 1e-12)
        x_an = jnp.maximum(g_an[sl], 1e-12)
        d_ap = x_ap * _rsqrt_newton(x_ap)
        d_an = x_an * _rsqrt_newton(x_an)
        return acc + jnp.maximum(d_ap - d_an + MARGIN, 0.0)

    acc = lax.fori_loop(0, N_VECS, loss_body, jnp.zeros((LANES,), jnp.float32))
    accv[...] = acc
    pltpu.sync_copy(accv, out_hbm.at[pl.ds(wid * LANES, LANES)])


_sc_pair_loss = pl.kernel(
    _sc_body,
    mesh=plsc.VectorSubcoreMesh(core_axis_name="c", subcore_axis_name="s"),
    compiler_params=pltpu.CompilerParams(use_tc_tiling_on_sc=False),
    out_type=jax.ShapeDtypeStruct((NW * LANES,), jnp.float32),
    scratch_types=[
        pltpu.VMEM((B_PER_W,), jnp.int32),
        pltpu.VMEM((B_PER_W,), jnp.int32),
        pltpu.VMEM((B_PER_W,), jnp.int32),
        pltpu.VMEM((B_PER_W,), jnp.int32),
        pltpu.VMEM((B_PER_W,), jnp.int32),
        pltpu.VMEM((B_PER_W,), jnp.float32),
        pltpu.VMEM((B_PER_W,), jnp.float32),
        pltpu.VMEM((LANES,), jnp.float32),
        pltpu.SemaphoreType.DMA,
    ],
)


def kernel(anchor, positive, negative, table):
    d2 = _tc_gram(table)
    d2_flat = d2.reshape(-1)
    parts = _sc_pair_loss(
        d2_flat, anchor.astype(jnp.int32), positive.astype(jnp.int32),
        negative.astype(jnp.int32))
    return jnp.sum(parts) / BATCH
